# Initial kernel scaffold; baseline (speedup 1.0000x reference)
import jax
import jax.numpy as jnp
from jax.experimental import pallas as pl


def kernel(input1, input2, c1, c2, c3, c4, g1, b1, g2, b2, g3, b3, g4, b4, w1, bb1, w2, bb2):
    raise NotImplementedError("write your pallas kernel here")


# speedup vs baseline: 144.9736x; 144.9736x over previous
import functools

import jax
import jax.numpy as jnp
from jax import lax
from jax.experimental import pallas as pl
from jax.experimental.pallas import tpu as pltpu

SHOT = 5
NC = 5
HW = 441
K = 3
CH = 64


def _conv(x, w):
    return lax.conv_general_dilated(
        x, w, (1, 1), ((1, 1), (1, 1)),
        dimension_numbers=('NCHW', 'OIHW', 'NCHW'))


def _bn(x, g, b):
    m = jnp.mean(x, axis=(0, 2, 3), keepdims=True)
    v = jnp.var(x, axis=(0, 2, 3), keepdims=True)
    return (x - m) / jnp.sqrt(v + 1e-5) * g.reshape(1, -1, 1, 1) + b.reshape(1, -1, 1, 1)


def _lrelu(x):
    return jnp.where(x >= 0, x, 0.2 * x)


def _pool(x):
    return lax.reduce_window(x, -jnp.inf, lax.max, (1, 1, 2, 2), (1, 1, 2, 2), 'VALID')


def _features(x, p):
    x = _pool(_lrelu(_bn(_conv(x, p['c1']), p['g1'], p['b1'])))
    x = _pool(_lrelu(_bn(_conv(x, p['c2']), p['g2'], p['b2'])))
    x = _lrelu(_bn(_conv(x, p['c3']), p['g3'], p['b3']))
    x = _lrelu(_bn(_conv(x, p['c4']), p['g4'], p['b4']))
    return x


def _prep_kernel(qs_ref, s_ref, w1bt_ref, bb1_ref,
                 sn_ref, b1eff_ref, sims_ref):
    qs = qs_ref[...]
    r2 = jnp.sum(qs * qs, axis=2, keepdims=True)
    qs_n = qs / jnp.sqrt(r2)
    qproto = jnp.mean(qs_n, axis=1)

    s = s_ref[...]
    protos = jnp.mean(s, axis=2)
    colnorm = jnp.sqrt(jnp.sum(s * s, axis=1, keepdims=True))
    sn_ref[...] = s / colnorm

    allp = protos.reshape(NC * CH, 1)
    wks = []
    for k in range(SHOT):
        pk = protos[k:k + 1, :]
        d = pk - allp
        nrm = jnp.sqrt(jnp.sum(d * d, axis=1, keepdims=True))
        mmd = jnp.mean(jnp.exp(-jnp.square(nrm / 2.0)))
        wks.append(1.0 - mmd)
    total_w = (wks[0] + wks[1] + wks[2] + wks[3] + wks[4]) * float(NC)
    sp = jnp.zeros((1, CH), dtype=jnp.float32)
    for k in range(SHOT):
        sp = sp + wks[k] * protos[k:k + 1, :]
    sp = sp / total_w

    for j in range(NC):
        b1eff_ref[j:j + 1, :] = bb1_ref[j:j + 1, :] + jnp.dot(
            sp, w1bt_ref[j], preferred_element_type=jnp.float32)

    qn = jnp.sqrt(jnp.sum(qproto * qproto, axis=1, keepdims=True))
    cols = []
    for j in range(NC):
        pj = protos[j:j + 1, :]
        num = jnp.sum(qproto * pj, axis=1, keepdims=True)
        pn = jnp.sqrt(jnp.sum(pj * pj))
        den = jnp.maximum(qn, 1e-8) * jnp.maximum(pn, 1e-8)
        cols.append(num / den)
    sims_ref[...] = jnp.concatenate(cols, axis=1)


def _main_kernel(qs_ref, sn_ref, w1at_ref, b1eff_ref, w2t_ref, bb2_ref,
                 out_ref):
    qs = qs_ref[0]
    r2 = jnp.sum(qs * qs, axis=1, keepdims=True)
    qsn = qs / jnp.sqrt(r2)
    b1eff = b1eff_ref[0]
    bb2 = bb2_ref[0]

    m = jnp.dot(qsn, sn_ref[0], preferred_element_type=jnp.float32)

    m1 = jnp.max(m, axis=1, keepdims=True)
    e1 = m == m1
    c1 = jnp.sum(e1.astype(jnp.float32), axis=1, keepdims=True)
    m_2 = jnp.where(e1, -3.0, m)
    m2 = jnp.max(m_2, axis=1, keepdims=True)
    e2 = m_2 == m2
    c2 = jnp.sum(e2.astype(jnp.float32), axis=1, keepdims=True)
    m_3 = jnp.where(e2, -3.0, m_2)
    m3 = jnp.max(m_3, axis=1, keepdims=True)
    k1 = jnp.minimum(c1, 3.0)
    k2 = jnp.clip(c2, 0.0, 3.0 - k1)
    k3 = jnp.maximum(3.0 - k1 - k2, 0.0)
    simv = m1 * k1 + m2 * k2 + m3 * k3

    h = jnp.maximum(
        jnp.dot(qsn, w1at_ref[0], preferred_element_type=jnp.float32)
        + b1eff, 0.0)
    o = jnp.dot(h, w2t_ref[0], preferred_element_type=jnp.float32) + bb2
    omax = jnp.max(o, axis=1, keepdims=True)
    e = jnp.exp(o - omax)
    z = jnp.sum(e, axis=1, keepdims=True)
    en = e / z
    cs = jnp.sum(en, axis=0, keepdims=True)
    r = jnp.dot(cs, simv, preferred_element_type=jnp.float32)
    out_ref[...] = r.reshape(1, 1, 1, 1)


def kernel(input1, input2, c1, c2, c3, c4, g1, b1, g2, b2, g3, b3, g4, b4,
           w1, bb1, w2, bb2):
    p = dict(c1=c1, c2=c2, c3=c3, c4=c4, g1=g1, b1=b1, g2=g2, b2=b2,
             g3=g3, b3=b3, g4=g4, b4=b4)
    q = _features(input1, p)
    B = q.shape[0]
    qs_raw = jnp.transpose(q.reshape(B, CH, HW), (0, 2, 1))

    s_list = []
    for i in range(NC):
        s = _features(input2[i], p)
        s = jnp.transpose(s, (1, 0, 2, 3)).reshape(CH, SHOT * HW)
        s_list.append(s)
    s_all = jnp.stack(s_list, axis=0)

    w1bt = jnp.transpose(w1[:, :, CH:], (0, 2, 1))
    w1at = jnp.transpose(w1[:, :, :CH], (0, 2, 1))
    w2t = jnp.transpose(w2, (0, 2, 1))

    sn, b1eff, sims = pl.pallas_call(
        _prep_kernel,
        out_shape=(
            jax.ShapeDtypeStruct((NC, CH, SHOT * HW), jnp.float32),
            jax.ShapeDtypeStruct((NC, 256), jnp.float32),
            jax.ShapeDtypeStruct((B, NC), jnp.float32),
        ),
    )(qs_raw, s_all, w1bt, bb1)

    sims_local = pl.pallas_call(
        _main_kernel,
        grid=(NC, B),
        in_specs=[
            pl.BlockSpec((1, HW, CH), lambda j, b: (b, 0, 0)),
            pl.BlockSpec((1, CH, SHOT * HW), lambda j, b: (j, 0, 0)),
            pl.BlockSpec((1, CH, 256), lambda j, b: (j, 0, 0)),
            pl.BlockSpec((1, 1, 256), lambda j, b: (j, 0, 0)),
            pl.BlockSpec((1, 256, HW), lambda j, b: (j, 0, 0)),
            pl.BlockSpec((1, 1, HW), lambda j, b: (j, 0, 0)),
        ],
        out_specs=pl.BlockSpec((1, 1, 1, 1), lambda j, b: (j, b, 0, 0)),
        out_shape=jax.ShapeDtypeStruct((NC, B, 1, 1), jnp.float32),
        compiler_params=pltpu.CompilerParams(
            dimension_semantics=("arbitrary", "arbitrary")),
    )(qs_raw, sn, w1at, b1eff[:, None, :], w2t, bb2[:, None, :])

    return (sims, jnp.transpose(sims_local.reshape(NC, B), (1, 0)))


# speedup vs baseline: 157.4423x; 1.0860x over previous
import functools

import jax
import jax.numpy as jnp
from jax import lax
from jax.experimental import pallas as pl
from jax.experimental.pallas import tpu as pltpu

SHOT = 5
NC = 5
HW = 441
K = 3
CH = 64


def _conv(x, w):
    return lax.conv_general_dilated(
        x, w, (1, 1), ((1, 1), (1, 1)),
        dimension_numbers=('NCHW', 'OIHW', 'NCHW'))


def _bn(x, g, b):
    m = jnp.mean(x, axis=(0, 2, 3), keepdims=True)
    v = jnp.var(x, axis=(0, 2, 3), keepdims=True)
    return (x - m) / jnp.sqrt(v + 1e-5) * g.reshape(1, -1, 1, 1) + b.reshape(1, -1, 1, 1)


def _lrelu(x):
    return jnp.where(x >= 0, x, 0.2 * x)


def _pool(x):
    return lax.reduce_window(x, -jnp.inf, lax.max, (1, 1, 2, 2), (1, 1, 2, 2), 'VALID')


def _features(x, p):
    x = _pool(_lrelu(_bn(_conv(x, p['c1']), p['g1'], p['b1'])))
    x = _pool(_lrelu(_bn(_conv(x, p['c2']), p['g2'], p['b2'])))
    x = _lrelu(_bn(_conv(x, p['c3']), p['g3'], p['b3']))
    x = _lrelu(_bn(_conv(x, p['c4']), p['g4'], p['b4']))
    return x


def _prep_kernel(qs_ref, s_ref, w1bt_ref, bb1_ref,
                 sn_ref, b1eff_ref, sims_ref):
    qs = qs_ref[...]
    r2 = jnp.sum(qs * qs, axis=2, keepdims=True)
    qs_n = qs / jnp.sqrt(r2)
    qproto = jnp.mean(qs_n, axis=1)

    s = s_ref[...]
    protos = jnp.mean(s, axis=2)
    colnorm = jnp.sqrt(jnp.sum(s * s, axis=1, keepdims=True))
    sn_ref[...] = s / colnorm

    allp = protos.reshape(NC * CH, 1)
    wks = []
    for k in range(SHOT):
        pk = protos[k:k + 1, :]
        d = pk - allp
        nrm = jnp.sqrt(jnp.sum(d * d, axis=1, keepdims=True))
        mmd = jnp.mean(jnp.exp(-jnp.square(nrm / 2.0)))
        wks.append(1.0 - mmd)
    total_w = (wks[0] + wks[1] + wks[2] + wks[3] + wks[4]) * float(NC)
    sp = jnp.zeros((1, CH), dtype=jnp.float32)
    for k in range(SHOT):
        sp = sp + wks[k] * protos[k:k + 1, :]
    sp = sp / total_w

    for j in range(NC):
        b1eff_ref[j:j + 1, :] = bb1_ref[j:j + 1, :] + jnp.dot(
            sp, w1bt_ref[j], preferred_element_type=jnp.float32)

    qn = jnp.sqrt(jnp.sum(qproto * qproto, axis=1, keepdims=True))
    cols = []
    for j in range(NC):
        pj = protos[j:j + 1, :]
        num = jnp.sum(qproto * pj, axis=1, keepdims=True)
        pn = jnp.sqrt(jnp.sum(pj * pj))
        den = jnp.maximum(qn, 1e-8) * jnp.maximum(pn, 1e-8)
        cols.append(num / den)
    sims_ref[...] = jnp.concatenate(cols, axis=1)


def _main_kernel(qs_ref, sn_ref, w1at_ref, b1eff_ref, w2t_ref, bb2_ref,
                 out_ref):
    qs = qs_ref[0]
    r2 = jnp.sum(qs * qs, axis=1, keepdims=True)
    qsn = qs / jnp.sqrt(r2)
    b1eff = b1eff_ref[0]
    bb2 = bb2_ref[0]

    m = jnp.dot(qsn.astype(jnp.bfloat16), sn_ref[0].astype(jnp.bfloat16),
                preferred_element_type=jnp.float32)

    m1 = jnp.max(m, axis=1, keepdims=True)
    m_2 = jnp.where(m == m1, -3.0, m)
    m2 = jnp.max(m_2, axis=1, keepdims=True)
    m_3 = jnp.where(m_2 == m2, -3.0, m_2)
    m3 = jnp.max(m_3, axis=1, keepdims=True)
    simv = m1 + m2 + m3

    h = jnp.maximum(
        jnp.dot(qsn, w1at_ref[0], preferred_element_type=jnp.float32)
        + b1eff, 0.0)
    o = jnp.dot(h, w2t_ref[0], preferred_element_type=jnp.float32) + bb2
    omax = jnp.max(o, axis=1, keepdims=True)
    e = jnp.exp(o - omax)
    z = jnp.sum(e, axis=1, keepdims=True)
    t = jnp.dot(e, simv, preferred_element_type=jnp.float32)
    r = jnp.sum(t / z)
    out_ref[...] = r.reshape(1, 1, 1, 1)


def kernel(input1, input2, c1, c2, c3, c4, g1, b1, g2, b2, g3, b3, g4, b4,
           w1, bb1, w2, bb2):
    p = dict(c1=c1, c2=c2, c3=c3, c4=c4, g1=g1, b1=b1, g2=g2, b2=b2,
             g3=g3, b3=b3, g4=g4, b4=b4)
    q = _features(input1, p)
    B = q.shape[0]
    qs_raw = jnp.transpose(q.reshape(B, CH, HW), (0, 2, 1))

    s_list = []
    for i in range(NC):
        s = _features(input2[i], p)
        s = jnp.transpose(s, (1, 0, 2, 3)).reshape(CH, SHOT * HW)
        s_list.append(s)
    s_all = jnp.stack(s_list, axis=0)

    w1bt = jnp.transpose(w1[:, :, CH:], (0, 2, 1))
    w1at = jnp.transpose(w1[:, :, :CH], (0, 2, 1))
    w2t = jnp.transpose(w2, (0, 2, 1))

    sn, b1eff, sims = pl.pallas_call(
        _prep_kernel,
        out_shape=(
            jax.ShapeDtypeStruct((NC, CH, SHOT * HW), jnp.float32),
            jax.ShapeDtypeStruct((NC, 256), jnp.float32),
            jax.ShapeDtypeStruct((B, NC), jnp.float32),
        ),
    )(qs_raw, s_all, w1bt, bb1)

    sims_local = pl.pallas_call(
        _main_kernel,
        grid=(NC, B),
        in_specs=[
            pl.BlockSpec((1, HW, CH), lambda j, b: (b, 0, 0)),
            pl.BlockSpec((1, CH, SHOT * HW), lambda j, b: (j, 0, 0)),
            pl.BlockSpec((1, CH, 256), lambda j, b: (j, 0, 0)),
            pl.BlockSpec((1, 1, 256), lambda j, b: (j, 0, 0)),
            pl.BlockSpec((1, 256, HW), lambda j, b: (j, 0, 0)),
            pl.BlockSpec((1, 1, HW), lambda j, b: (j, 0, 0)),
        ],
        out_specs=pl.BlockSpec((1, 1, 1, 1), lambda j, b: (j, b, 0, 0)),
        out_shape=jax.ShapeDtypeStruct((NC, B, 1, 1), jnp.float32),
        compiler_params=pltpu.CompilerParams(
            dimension_semantics=("arbitrary", "arbitrary")),
    )(qs_raw, sn, w1at, b1eff[:, None, :], w2t, bb2[:, None, :])

    return (sims, jnp.transpose(sims_local.reshape(NC, B), (1, 0)))


# speedup vs baseline: 182.7366x; 1.1607x over previous
import functools

import jax
import jax.numpy as jnp
from jax import lax
from jax.experimental import pallas as pl
from jax.experimental.pallas import tpu as pltpu

SHOT = 5
NC = 5
HW = 441
K = 3
CH = 64


_GROUP_OF_IMG = [0] * 75 + [1 + i // SHOT for i in range(NC * SHOT)]


def _lrelu_affine(x, sc, sh):
    z = x * sc + sh
    return jnp.where(z >= 0, z, 0.2 * z)


def _group_affine(y, g, b):
    yq = y[:75]
    ys = y[75:].reshape(NC, SHOT, y.shape[1], y.shape[2], CH)
    mq = jnp.mean(yq, axis=(0, 1, 2))
    sq = jnp.mean(jnp.square(yq), axis=(0, 1, 2))
    mc = jnp.mean(ys, axis=(1, 2, 3))
    sc = jnp.mean(jnp.square(ys), axis=(1, 2, 3))
    m = jnp.concatenate([mq[None, :], mc], axis=0)
    s = jnp.concatenate([sq[None, :], sc], axis=0)
    v = s - jnp.square(m)
    scale = g[None, :] / jnp.sqrt(v + 1e-5)
    shift = b[None, :] - m * scale
    return scale, shift


def _bn_lrelu(y, g, b, pool):
    scale, shift = _group_affine(y, g, b)
    if pool:
        n, h, w, c = y.shape
        y = jnp.max(y.reshape(n, h // 2, 2, w // 2, 2, c), axis=(2, 4))
    gidx = jnp.asarray(_GROUP_OF_IMG, dtype=jnp.int32)
    return _lrelu_affine(y, scale[gidx][:, None, None, :],
                         shift[gidx][:, None, None, :])


def _conv_nhwc(x, w):
    return lax.conv_general_dilated(
        x, jnp.transpose(w, (2, 3, 1, 0)), (1, 1), ((1, 1), (1, 1)),
        dimension_numbers=('NHWC', 'HWIO', 'NHWC'))


def _features_all(x, p):
    x = _bn_lrelu(_conv_nhwc(x, p['c1']), p['g1'], p['b1'], pool=True)
    x = _bn_lrelu(_conv_nhwc(x, p['c2']), p['g2'], p['b2'], pool=True)
    x = _bn_lrelu(_conv_nhwc(x, p['c3']), p['g3'], p['b3'], pool=False)
    y4 = _conv_nhwc(x, p['c4'])
    sc4, sh4 = _group_affine(y4, p['g4'], p['b4'])
    return y4, sc4, sh4


def _prep_kernel(qs_ref, s_ref, scq_ref, shq_ref, scs_ref, shs_ref,
                 w1bt_ref, bb1_ref, sn_ref, b1eff_ref, sims_ref):
    qs = _lrelu_affine(qs_ref[...], scq_ref[...], shq_ref[...])
    r2 = jnp.sum(qs * qs, axis=2, keepdims=True)
    qs_n = qs / jnp.sqrt(r2)
    qproto = jnp.mean(qs_n, axis=1)

    s = _lrelu_affine(s_ref[...], scs_ref[...], shs_ref[...])
    protos = jnp.mean(s, axis=2)
    colnorm = jnp.sqrt(jnp.sum(s * s, axis=1, keepdims=True))
    sn_ref[...] = (s / colnorm).astype(jnp.bfloat16)

    allp = protos.reshape(NC * CH, 1)
    wks = []
    for k in range(SHOT):
        pk = protos[k:k + 1, :]
        d = pk - allp
        nrm = jnp.sqrt(jnp.sum(d * d, axis=1, keepdims=True))
        mmd = jnp.mean(jnp.exp(-jnp.square(nrm / 2.0)))
        wks.append(1.0 - mmd)
    total_w = (wks[0] + wks[1] + wks[2] + wks[3] + wks[4]) * float(NC)
    sp = jnp.zeros((1, CH), dtype=jnp.float32)
    for k in range(SHOT):
        sp = sp + wks[k] * protos[k:k + 1, :]
    sp = sp / total_w

    for j in range(NC):
        b1eff_ref[j:j + 1, :] = bb1_ref[j:j + 1, :] + jnp.dot(
            sp, w1bt_ref[j], preferred_element_type=jnp.float32)

    qn = jnp.sqrt(jnp.sum(qproto * qproto, axis=1, keepdims=True))
    cols = []
    for j in range(NC):
        pj = protos[j:j + 1, :]
        num = jnp.sum(qproto * pj, axis=1, keepdims=True)
        pn = jnp.sqrt(jnp.sum(pj * pj))
        den = jnp.maximum(qn, 1e-8) * jnp.maximum(pn, 1e-8)
        cols.append(num / den)
    sims_ref[...] = jnp.concatenate(cols, axis=1)


def _main_kernel(qs_ref, scq_ref, shq_ref, sn_ref, w1at_ref, b1eff_ref,
                 w2t_ref, bb2_ref, out_ref):
    qs = _lrelu_affine(qs_ref[0], scq_ref[0], shq_ref[0])
    r2 = jnp.sum(qs * qs, axis=1, keepdims=True)
    qsn = qs / jnp.sqrt(r2)
    b1eff = b1eff_ref[0]
    bb2 = bb2_ref[0]

    qsnh = qsn.astype(jnp.bfloat16)
    m = jnp.dot(qsnh, sn_ref[0],
                preferred_element_type=jnp.float32)

    m1 = jnp.max(m, axis=1, keepdims=True)
    m_2 = jnp.where(m == m1, -3.0, m)
    m2 = jnp.max(m_2, axis=1, keepdims=True)
    m_3 = jnp.where(m_2 == m2, -3.0, m_2)
    m3 = jnp.max(m_3, axis=1, keepdims=True)
    simv = m1 + m2 + m3

    h = jnp.maximum(
        jnp.dot(qsnh, w1at_ref[0], preferred_element_type=jnp.float32)
        + b1eff, 0.0)
    o = jnp.dot(h.astype(jnp.bfloat16), w2t_ref[0],
                preferred_element_type=jnp.float32) + bb2
    omax = jnp.max(o, axis=1, keepdims=True)
    e = jnp.exp(o - omax)
    z = jnp.sum(e, axis=1, keepdims=True)
    t = jnp.dot(e, simv, preferred_element_type=jnp.float32)
    r = jnp.sum(t / z)
    out_ref[...] = r.reshape(1, 1, 1, 1)


def kernel(input1, input2, c1, c2, c3, c4, g1, b1, g2, b2, g3, b3, g4, b4,
           w1, bb1, w2, bb2):
    p = dict(c1=c1, c2=c2, c3=c3, c4=c4, g1=g1, b1=b1, g2=g2, b2=b2,
             g3=g3, b3=b3, g4=g4, b4=b4)
    B = input1.shape[0]
    x_all = jnp.concatenate([
        jnp.transpose(input1, (0, 2, 3, 1)),
        jnp.transpose(input2.reshape(NC * SHOT, 3, 84, 84), (0, 2, 3, 1)),
    ], axis=0)
    y4, sc4, sh4 = _features_all(x_all, p)
    y4q = y4[:B].reshape(B, HW, CH)
    y4s = jnp.transpose(
        y4[B:].reshape(NC, SHOT * HW, CH), (0, 2, 1))
    scq = sc4[0:1][None]
    shq = sh4[0:1][None]
    scs = sc4[1:][:, :, None]
    shs = sh4[1:][:, :, None]

    w1bt = jnp.transpose(w1[:, :, CH:], (0, 2, 1))
    w1at = jnp.transpose(w1[:, :, :CH], (0, 2, 1)).astype(jnp.bfloat16)
    w2t = jnp.transpose(w2, (0, 2, 1)).astype(jnp.bfloat16)

    sn, b1eff, sims = pl.pallas_call(
        _prep_kernel,
        out_shape=(
            jax.ShapeDtypeStruct((NC, CH, SHOT * HW), jnp.bfloat16),
            jax.ShapeDtypeStruct((NC, 256), jnp.float32),
            jax.ShapeDtypeStruct((B, NC), jnp.float32),
        ),
    )(y4q, y4s, scq, shq, scs, shs, w1bt, bb1)

    sims_local = pl.pallas_call(
        _main_kernel,
        grid=(NC, B),
        in_specs=[
            pl.BlockSpec((1, HW, CH), lambda j, b: (b, 0, 0)),
            pl.BlockSpec((1, 1, CH), lambda j, b: (0, 0, 0)),
            pl.BlockSpec((1, 1, CH), lambda j, b: (0, 0, 0)),
            pl.BlockSpec((1, CH, SHOT * HW), lambda j, b: (j, 0, 0)),
            pl.BlockSpec((1, CH, 256), lambda j, b: (j, 0, 0)),
            pl.BlockSpec((1, 1, 256), lambda j, b: (j, 0, 0)),
            pl.BlockSpec((1, 256, HW), lambda j, b: (j, 0, 0)),
            pl.BlockSpec((1, 1, HW), lambda j, b: (j, 0, 0)),
        ],
        out_specs=pl.BlockSpec((1, 1, 1, 1), lambda j, b: (j, b, 0, 0)),
        out_shape=jax.ShapeDtypeStruct((NC, B, 1, 1), jnp.float32),
        compiler_params=pltpu.CompilerParams(
            dimension_semantics=("arbitrary", "arbitrary")),
    )(y4q, scq, shq, sn, w1at, b1eff[:, None, :], w2t, bb2[:, None, :])

    return (sims, jnp.transpose(sims_local.reshape(NC, B), (1, 0)))
